# baseline (speedup 1.0000x reference)
import jax
import jax.numpy as jnp
from jax.experimental import pallas as pl
from jax.experimental.pallas import tpu as pltpu

E = 64
K = 2
D = 768
F = 1536
T = 128
NBUF = 3
DH = D // 2
FH = F // 2


def _moe_body(x_ref, Wg_ref, bg_ref, b1_ref, b2_ref, W1_hbm, W2_hbm,
              out_ref, w_ref, acc_ref,
              w1a, w1b, w2a, w2b, s1a, s1b, s2a, s2b):
    e = pl.program_id(0)

    def issue(idx, slot):
        pltpu.make_async_copy(W1_hbm.at[idx, pl.ds(0, DH)], w1a.at[slot],
                              s1a.at[slot]).start()
        pltpu.make_async_copy(W1_hbm.at[idx, pl.ds(DH, DH)], w1b.at[slot],
                              s1b.at[slot]).start()
        pltpu.make_async_copy(W2_hbm.at[idx, pl.ds(0, FH)], w2a.at[slot],
                              s2a.at[slot]).start()
        pltpu.make_async_copy(W2_hbm.at[idx, pl.ds(FH, FH)], w2b.at[slot],
                              s2b.at[slot]).start()

    @pl.when(e == 0)
    def _prologue():
        for i in range(NBUF):
            issue(i, i)
        xb = x_ref[:]
        logits = (jnp.dot(xb, Wg_ref[:], preferred_element_type=jnp.float32)
                  + bg_ref[0, :])
        probs = jax.nn.softmax(logits, axis=-1)
        eidx = jax.lax.broadcasted_iota(jnp.int32, (T, E), 1)
        m1 = jnp.max(probs, axis=1, keepdims=True)
        i1 = jnp.argmax(probs, axis=1)[:, None]
        probs2 = jnp.where(eidx == i1, -jnp.inf, probs)
        m2 = jnp.max(probs2, axis=1, keepdims=True)
        i2 = jnp.argmax(probs2, axis=1)[:, None]
        denom = m1 + m2
        w = (jnp.where(eidx == i1, m1, 0.0)
             + jnp.where(eidx == i2, m2, 0.0)) / denom
        w_ref[:] = w
        acc_ref[:] = jnp.zeros_like(acc_ref)

    @pl.when((e > 0) & (e + NBUF - 1 < E))
    def _prefetch():
        nxt = e + NBUF - 1
        issue(nxt, jax.lax.rem(nxt, NBUF))

    slot = jax.lax.rem(e, NBUF)
    pltpu.make_async_copy(W1_hbm.at[e, pl.ds(0, DH)], w1a.at[slot],
                          s1a.at[slot]).wait()
    pltpu.make_async_copy(W1_hbm.at[e, pl.ds(DH, DH)], w1b.at[slot],
                          s1b.at[slot]).wait()
    pltpu.make_async_copy(W2_hbm.at[e, pl.ds(0, FH)], w2a.at[slot],
                          s2a.at[slot]).wait()
    pltpu.make_async_copy(W2_hbm.at[e, pl.ds(FH, FH)], w2b.at[slot],
                          s2b.at[slot]).wait()

    xb = x_ref[:]
    h = jnp.maximum(
        jnp.dot(xb[:, :DH], w1a[slot], preferred_element_type=jnp.float32)
        + jnp.dot(xb[:, DH:], w1b[slot], preferred_element_type=jnp.float32)
        + b1_ref[0, 0, :], 0.0)
    o = (jnp.dot(h[:, :FH], w2a[slot], preferred_element_type=jnp.float32)
         + jnp.dot(h[:, FH:], w2b[slot], preferred_element_type=jnp.float32))
    eidx = jax.lax.broadcasted_iota(jnp.int32, (T, E), 1)
    wcol = jnp.sum(jnp.where(eidx == e, w_ref[:], 0.0), axis=1, keepdims=True)
    acc_ref[:] += wcol * o

    @pl.when(e == E - 1)
    def _finish():
        out_ref[:] = acc_ref[:] + jnp.dot(
            w_ref[:], b2_ref[:], preferred_element_type=jnp.float32)


def kernel(x, Wg, bg, W1, b1, W2, b2):
    B, S, _ = x.shape
    xf = x.reshape(T, D)
    bg2 = bg.reshape(1, E)
    b1r = b1.reshape(E, 1, F)
    out = pl.pallas_call(
        _moe_body,
        grid=(E,),
        in_specs=[
            pl.BlockSpec((T, D), lambda e: (0, 0)),
            pl.BlockSpec((D, E), lambda e: (0, 0)),
            pl.BlockSpec((1, E), lambda e: (0, 0)),
            pl.BlockSpec((1, 1, F), lambda e: (e, 0, 0)),
            pl.BlockSpec((E, D), lambda e: (0, 0)),
            pl.BlockSpec(memory_space=pltpu.MemorySpace.HBM),
            pl.BlockSpec(memory_space=pltpu.MemorySpace.HBM),
        ],
        out_specs=pl.BlockSpec((T, D), lambda e: (0, 0)),
        out_shape=jax.ShapeDtypeStruct((T, D), jnp.float32),
        scratch_shapes=[
            pltpu.VMEM((T, E), jnp.float32),
            pltpu.VMEM((T, D), jnp.float32),
            pltpu.VMEM((NBUF, DH, F), jnp.float32),
            pltpu.VMEM((NBUF, DH, F), jnp.float32),
            pltpu.VMEM((NBUF, FH, D), jnp.float32),
            pltpu.VMEM((NBUF, FH, D), jnp.float32),
            pltpu.SemaphoreType.DMA((NBUF,)),
            pltpu.SemaphoreType.DMA((NBUF,)),
            pltpu.SemaphoreType.DMA((NBUF,)),
            pltpu.SemaphoreType.DMA((NBUF,)),
        ],
    )(xf, Wg, bg2, b1r, b2, W1, W2)
    return out.reshape(B, S, D)
